# Initial kernel scaffold; baseline (speedup 1.0000x reference)
import jax
import jax.numpy as jnp
from jax.experimental import pallas as pl


def kernel(x, embed_table, W, b):
    raise NotImplementedError("write your pallas kernel here")


# speedup vs baseline: 2.2587x; 2.2587x over previous
import functools

import jax
import jax.numpy as jnp
from jax import lax
from jax.experimental import pallas as pl
from jax.experimental.pallas import tpu as pltpu
from jax.experimental.pallas import tpu_sc as plsc

_DIM = 32
_VPAD = 128
_C = 128
_K = 8
_NC = 2
_NS = 16
_NW = _NC * _NS


@functools.cache
def _make_sc_kernel(n_rows):
    rows_per_w = n_rows // _NW
    steps = rows_per_w // _K
    mesh = plsc.VectorSubcoreMesh(core_axis_name="c", subcore_axis_name="s")

    @functools.partial(
        pl.kernel,
        mesh=mesh,
        compiler_params=pltpu.CompilerParams(
            needs_layout_passes=False, use_tc_tiling_on_sc=False),
        out_type=jax.ShapeDtypeStruct((n_rows, _C, _DIM), jnp.float32),
        scratch_types=[
            pltpu.VMEM((_VPAD // 4, 128), jnp.float32),
            pltpu.VMEM((_DIM * _DIM // 128, 128), jnp.float32),
            pltpu.VMEM((128,), jnp.float32),
            pltpu.VMEM((_VPAD // 4, 128), jnp.float32),
            pltpu.VMEM((_K, _C), jnp.int32),
            pltpu.VMEM((_K, _C, _DIM), jnp.float32),
        ],
    )
    def sc_kernel(idx_hbm, e_hbm, w_hbm, b_hbm, out_hbm,
                  e_v, w_v, b_v, t_v, idx_v, out_v):
        pltpu.sync_copy(e_hbm, e_v)
        pltpu.sync_copy(w_hbm, w_v)
        pltpu.sync_copy(b_hbm, b_v)

        b0 = b_v[pl.ds(0, 16)]
        b1 = b_v[pl.ds(16, 16)]

        def build_row(v, carry):
            r = v >> 2
            c = (v & 3) * 32
            rowv = jnp.full((16,), r, dtype=jnp.int32)
            acc0, acc1 = b0, b1
            for k in range(_DIM):
                ek = plsc.load_gather(
                    e_v, [rowv, jnp.full((16,), c + k, dtype=jnp.int32)])
                wf = k * _DIM
                w0 = w_v[wf // 128, pl.ds(wf % 128, 16)]
                w1 = w_v[wf // 128, pl.ds(wf % 128 + 16, 16)]
                acc0 = acc0 + ek * w0
                acc1 = acc1 + ek * w1
            t_v[r, pl.ds(c, 16)] = acc0
            t_v[r, pl.ds(c + 16, 16)] = acc1
            return carry

        lax.fori_loop(0, _VPAD, build_row, 0)

        wid = lax.axis_index("s") * _NC + lax.axis_index("c")
        row0 = wid * rows_per_w
        lane = lax.iota(jnp.int32, 16)

        def step(s, carry):
            r = row0 + s * _K
            pltpu.sync_copy(idx_hbm.at[pl.ds(r, _K)], idx_v)

            def grp(i, c2):
                iv = idx_v[i >> 3, pl.ds((i & 7) * 16, 16)]
                base = iv * _DIM
                t = i * 16 + lane
                d0 = t >> 7
                d1 = t & 127
                for j in range(_DIM):
                    addr = base + j
                    g = plsc.load_gather(t_v, [addr >> 7, addr & 127])
                    plsc.store_scatter(
                        out_v, [d0, d1, jnp.full((16,), j, dtype=jnp.int32)], g)
                return c2

            lax.fori_loop(0, _K * (_C // 16), grp, 0)
            pltpu.sync_copy(out_v, out_hbm.at[pl.ds(r, _K)])
            return carry

        lax.fori_loop(0, steps, step, 0)

    return sc_kernel


def kernel(x, embed_table, W, b):
    bsz, hist = x.shape
    n = bsz * hist
    idx = x.reshape(n // _C, _C).astype(jnp.int32)
    e2 = jnp.pad(embed_table.astype(jnp.float32),
                 ((0, _VPAD - embed_table.shape[0]), (0, 0))).reshape(-1, 128)
    w2 = W.astype(jnp.float32).T.reshape(-1, 128)
    b2 = jnp.pad(b.astype(jnp.float32), (0, 128 - _DIM))
    out = _make_sc_kernel(n // _C)(idx, e2, w2, b2)
    return out.reshape(bsz, hist, _DIM)


# speedup vs baseline: 5.3579x; 2.3721x over previous
import functools

import jax
import jax.numpy as jnp
from jax import lax
from jax.experimental import pallas as pl
from jax.experimental.pallas import tpu as pltpu
from jax.experimental.pallas import tpu_sc as plsc

_DIM = 32
_VPAD = 128
_C = 128
_K = 8
_NC = 2
_NS = 16
_NW = _NC * _NS


@functools.cache
def _make_sc_kernel(n_rows):
    rows_per_w = n_rows // _NW
    steps = rows_per_w // _K
    mesh = plsc.VectorSubcoreMesh(core_axis_name="c", subcore_axis_name="s")

    @functools.partial(
        pl.kernel,
        mesh=mesh,
        compiler_params=pltpu.CompilerParams(
            needs_layout_passes=False, use_tc_tiling_on_sc=False),
        out_type=jax.ShapeDtypeStruct((n_rows, _C, _DIM), jnp.float32),
        scratch_types=[
            pltpu.VMEM((_VPAD // 4, 128), jnp.float32),
            pltpu.VMEM((_DIM * _DIM // 128, 128), jnp.float32),
            pltpu.VMEM((128,), jnp.float32),
            pltpu.VMEM((_VPAD * _DIM,), jnp.float32),
            pltpu.VMEM((_K, _C), jnp.int32),
            pltpu.VMEM((_K, _C, _DIM), jnp.float32),
        ],
    )
    def sc_kernel(idx_hbm, e_hbm, w_hbm, b_hbm, out_hbm,
                  e_v, w_v, b_v, t1, idx_v, out_v):
        pltpu.sync_copy(e_hbm, e_v)
        pltpu.sync_copy(w_hbm, w_v)
        pltpu.sync_copy(b_hbm, b_v)

        b0 = b_v[pl.ds(0, 16)]
        b1 = b_v[pl.ds(16, 16)]

        def build_row(v, carry):
            r = v >> 2
            c = (v & 3) * 32
            rowv = jnp.full((16,), r, dtype=jnp.int32)
            acc0, acc1 = b0, b1
            for k in range(_DIM):
                ek = plsc.load_gather(
                    e_v, [rowv, jnp.full((16,), c + k, dtype=jnp.int32)])
                wf = k * _DIM
                w0 = w_v[wf // 128, pl.ds(wf % 128, 16)]
                w1 = w_v[wf // 128, pl.ds(wf % 128 + 16, 16)]
                acc0 = acc0 + ek * w0
                acc1 = acc1 + ek * w1
            t1[pl.ds(v * _DIM, 16)] = acc0
            t1[pl.ds(v * _DIM + 16, 16)] = acc1
            return carry

        lax.fori_loop(0, _VPAD, build_row, 0)

        wid = lax.axis_index("s") * _NC + lax.axis_index("c")
        row0 = wid * rows_per_w

        def step(s, carry):
            r = row0 + s * _K
            pltpu.sync_copy(idx_hbm.at[pl.ds(r, _K)], idx_v)

            def grp(i, c2):
                iv = idx_v[i >> 3, pl.ds((i & 7) * 16, 16)]
                d0 = i >> 3
                d1 = (i & 7) * 16
                for l in range(16):
                    base = iv[l] * _DIM
                    g0 = t1[pl.ds(base, 16)]
                    g1 = t1[pl.ds(base + 16, 16)]
                    out_v[d0, d1 + l, pl.ds(0, 16)] = g0
                    out_v[d0, d1 + l, pl.ds(16, 16)] = g1
                return c2

            lax.fori_loop(0, _K * (_C // 16), grp, 0)
            pltpu.sync_copy(out_v, out_hbm.at[pl.ds(r, _K)])
            return carry

        lax.fori_loop(0, steps, step, 0)

    return sc_kernel


def kernel(x, embed_table, W, b):
    bsz, hist = x.shape
    n = bsz * hist
    idx = x.reshape(n // _C, _C).astype(jnp.int32)
    e2 = jnp.pad(embed_table.astype(jnp.float32),
                 ((0, _VPAD - embed_table.shape[0]), (0, 0))).reshape(-1, 128)
    w2 = W.astype(jnp.float32).T.reshape(-1, 128)
    b2 = jnp.pad(b.astype(jnp.float32), (0, 128 - _DIM))
    out = _make_sc_kernel(n // _C)(idx, e2, w2, b2)
    return out.reshape(bsz, hist, _DIM)


# speedup vs baseline: 6.3125x; 1.1782x over previous
import functools

import jax
import jax.numpy as jnp
from jax import lax
from jax.experimental import pallas as pl
from jax.experimental.pallas import tpu as pltpu
from jax.experimental.pallas import tpu_sc as plsc

_DIM = 32
_VPAD = 128
_C = 128
_K = 8
_NC = 2
_NS = 16
_NW = _NC * _NS


@functools.cache
def _make_sc_kernel(n_rows):
    rows_per_w = n_rows // _NW
    steps = rows_per_w // _K
    mesh = plsc.VectorSubcoreMesh(core_axis_name="c", subcore_axis_name="s")

    @functools.partial(
        pl.kernel,
        mesh=mesh,
        compiler_params=pltpu.CompilerParams(
            needs_layout_passes=False, use_tc_tiling_on_sc=False),
        out_type=jax.ShapeDtypeStruct((n_rows, _C, _DIM), jnp.float32),
        scratch_types=[
            pltpu.VMEM((_VPAD // 4, 128), jnp.float32),
            pltpu.VMEM((_DIM * _DIM // 128, 128), jnp.float32),
            pltpu.VMEM((128,), jnp.float32),
            pltpu.VMEM((_VPAD, _DIM), jnp.float32),
            pltpu.VMEM_SHARED((_VPAD, _DIM), jnp.float32),
            pltpu.VMEM((_K, _C), jnp.int32),
            pltpu.VMEM((_K, _C, _DIM), jnp.float32),
            pltpu.SemaphoreType.DMA,
        ],
    )
    def sc_kernel(idx_hbm, e_hbm, w_hbm, b_hbm, out_hbm,
                  e_v, w_v, b_v, t2, t_s, idx_v, out_v, sem):
        pltpu.sync_copy(e_hbm, e_v)
        pltpu.sync_copy(w_hbm, w_v)
        pltpu.sync_copy(b_hbm, b_v)

        b0 = b_v[pl.ds(0, 16)]
        b1 = b_v[pl.ds(16, 16)]

        def build_row(v, carry):
            r = v >> 2
            c = (v & 3) * 32
            rowv = jnp.full((16,), r, dtype=jnp.int32)
            acc0, acc1 = b0, b1
            for k in range(_DIM):
                ek = plsc.load_gather(
                    e_v, [rowv, jnp.full((16,), c + k, dtype=jnp.int32)])
                wf = k * _DIM
                w0 = w_v[wf // 128, pl.ds(wf % 128, 16)]
                w1 = w_v[wf // 128, pl.ds(wf % 128 + 16, 16)]
                acc0 = acc0 + ek * w0
                acc1 = acc1 + ek * w1
            t2[v, pl.ds(0, 16)] = acc0
            t2[v, pl.ds(16, 16)] = acc1
            return carry

        lax.fori_loop(0, _VPAD, build_row, 0)

        @pl.when(lax.axis_index("s") == 0)
        def _publish():
            pltpu.sync_copy(t2, t_s)

        plsc.subcore_barrier()

        wid = lax.axis_index("s") * _NC + lax.axis_index("c")
        row0 = wid * rows_per_w

        def step(s, carry):
            r = row0 + s * _K
            pltpu.sync_copy(idx_hbm.at[pl.ds(r, _K)], idx_v)
            copies = [
                pltpu.async_copy(t_s.at[idx_v.at[j]], out_v.at[j], sem)
                for j in range(_K)
            ]
            for cp in copies:
                cp.wait()
            pltpu.sync_copy(out_v, out_hbm.at[pl.ds(r, _K)])
            return carry

        lax.fori_loop(0, steps, step, 0)

    return sc_kernel


def kernel(x, embed_table, W, b):
    bsz, hist = x.shape
    n = bsz * hist
    idx = x.reshape(n // _C, _C).astype(jnp.int32)
    e2 = jnp.pad(embed_table.astype(jnp.float32),
                 ((0, _VPAD - embed_table.shape[0]), (0, 0))).reshape(-1, 128)
    w2 = W.astype(jnp.float32).T.reshape(-1, 128)
    b2 = jnp.pad(b.astype(jnp.float32), (0, 128 - _DIM))
    out = _make_sc_kernel(n // _C)(idx, e2, w2, b2)
    return out.reshape(bsz, hist, _DIM)


# speedup vs baseline: 6.8965x; 1.0925x over previous
import functools

import jax
import jax.numpy as jnp
from jax import lax
from jax.experimental import pallas as pl
from jax.experimental.pallas import tpu as pltpu
from jax.experimental.pallas import tpu_sc as plsc

_DIM = 32
_VPAD = 128
_C = 128
_K = 8
_NC = 2
_NS = 16
_NW = _NC * _NS


@functools.cache
def _make_sc_kernel(n_rows):
    rows_per_w = n_rows // _NW
    steps = rows_per_w // _K
    mesh = plsc.VectorSubcoreMesh(core_axis_name="c", subcore_axis_name="s")

    @functools.partial(
        pl.kernel,
        mesh=mesh,
        compiler_params=pltpu.CompilerParams(
            needs_layout_passes=False, use_tc_tiling_on_sc=False),
        out_type=jax.ShapeDtypeStruct((n_rows, _C, _DIM), jnp.float32),
        scratch_types=[
            pltpu.VMEM((_VPAD // 4, 128), jnp.float32),
            pltpu.VMEM((_DIM * _DIM // 128, 128), jnp.float32),
            pltpu.VMEM((128,), jnp.float32),
            pltpu.VMEM((_VPAD, _DIM), jnp.float32),
            pltpu.VMEM_SHARED((_VPAD, _DIM), jnp.float32),
            pltpu.VMEM((2, _K, _C), jnp.int32),
            pltpu.VMEM((2, _K, _C, _DIM), jnp.float32),
            pltpu.SemaphoreType.DMA,
            pltpu.SemaphoreType.DMA,
            pltpu.SemaphoreType.DMA,
        ],
    )
    def sc_kernel(idx_hbm, e_hbm, w_hbm, b_hbm, out_hbm,
                  e_v, w_v, b_v, t2, t_s, idx_v, out_v, sem_i, sem_g, sem_o):
        pltpu.sync_copy(e_hbm, e_v)
        pltpu.sync_copy(w_hbm, w_v)
        pltpu.sync_copy(b_hbm, b_v)

        b0 = b_v[pl.ds(0, 16)]
        b1 = b_v[pl.ds(16, 16)]

        def build_row(v, carry):
            r = v >> 2
            c = (v & 3) * 32
            rowv = jnp.full((16,), r, dtype=jnp.int32)
            acc0, acc1 = b0, b1
            for k in range(_DIM):
                ek = plsc.load_gather(
                    e_v, [rowv, jnp.full((16,), c + k, dtype=jnp.int32)])
                wf = k * _DIM
                w0 = w_v[wf // 128, pl.ds(wf % 128, 16)]
                w1 = w_v[wf // 128, pl.ds(wf % 128 + 16, 16)]
                acc0 = acc0 + ek * w0
                acc1 = acc1 + ek * w1
            t2[v, pl.ds(0, 16)] = acc0
            t2[v, pl.ds(16, 16)] = acc1
            return carry

        lax.fori_loop(0, _VPAD, build_row, 0)

        @pl.when(lax.axis_index("s") == 0)
        def _publish():
            pltpu.sync_copy(t2, t_s)

        plsc.subcore_barrier()

        wid = lax.axis_index("s") * _NC + lax.axis_index("c")
        row0 = wid * rows_per_w

        def fire_idx(s, p):
            r = row0 + s * _K
            pltpu.async_copy(idx_hbm.at[pl.ds(r, _K)], idx_v.at[p], sem_i)

        def wait_idx(p):
            pltpu.make_async_copy(
                idx_hbm.at[pl.ds(row0, _K)], idx_v.at[p], sem_i).wait()

        def wait_out():
            pltpu.make_async_copy(
                out_v.at[0], out_hbm.at[pl.ds(row0, _K)], sem_o).wait()

        fire_idx(0, 0)

        def outer(o, carry):
            for p in range(2):
                s = o * 2 + p
                wait_idx(p)

                @pl.when(s + 1 < steps)
                def _prefetch():
                    fire_idx(s + 1, 1 - p)

                @pl.when(s >= 2)
                def _reclaim():
                    wait_out()

                copies = [
                    pltpu.async_copy(
                        t_s.at[idx_v.at[p].at[j]], out_v.at[p].at[j], sem_g)
                    for j in range(_K)
                ]
                for cp in copies:
                    cp.wait()
                pltpu.async_copy(
                    out_v.at[p], out_hbm.at[pl.ds(row0 + s * _K, _K)], sem_o)
            return carry

        lax.fori_loop(0, steps // 2, outer, 0)
        wait_out()
        wait_out()

    return sc_kernel


def kernel(x, embed_table, W, b):
    bsz, hist = x.shape
    n = bsz * hist
    idx = x.reshape(n // _C, _C).astype(jnp.int32)
    e2 = jnp.pad(embed_table.astype(jnp.float32),
                 ((0, _VPAD - embed_table.shape[0]), (0, 0))).reshape(-1, 128)
    w2 = W.astype(jnp.float32).T.reshape(-1, 128)
    b2 = jnp.pad(b.astype(jnp.float32), (0, 128 - _DIM))
    out = _make_sc_kernel(n // _C)(idx, e2, w2, b2)
    return out.reshape(bsz, hist, _DIM)
